# Initial kernel scaffold; baseline (speedup 1.0000x reference)
import jax
import jax.numpy as jnp
from jax.experimental import pallas as pl


def kernel(x, edge_index, batch, W1_0, b1_0, W2_0, b2_0, W1_1, b1_1, W2_1, b2_1, W1_2, b1_2, W2_2, b2_2, W_jk, b_jk, Wc1, bc1, gamma, beta, Wc2, bc2):
    raise NotImplementedError("write your pallas kernel here")


# speedup vs baseline: 3.2864x; 3.2864x over previous
import functools

import jax
import jax.numpy as jnp
from jax import lax
from jax.experimental import pallas as pl
from jax.experimental.pallas import tpu as pltpu
from jax.experimental.pallas import tpu_sc as plsc

N = 10000
E = 320000
H = 128
G = 128

NC = 2
NS = 16
NW = NC * NS
EPW = E // NW
CH = 80
NCHUNK = EPW // CH
RPT = 632
NP = NS * RPT

R = 1000
NBLK = N // R


@functools.partial(
    pl.kernel,
    out_type=jax.ShapeDtypeStruct((NC, NP, H), jnp.float32),
    mesh=plsc.VectorSubcoreMesh(core_axis_name="c", subcore_axis_name="s"),
    scratch_types=[
        pltpu.VMEM_SHARED((NP, H), jnp.float32),
        pltpu.VMEM((CH,), jnp.int32),
        pltpu.VMEM((CH,), jnp.int32),
        pltpu.VMEM((CH, H), jnp.float32),
        pltpu.SemaphoreType.DMA,
    ],
)
def _mp(src_hbm, dst_hbm, zer_hbm, h_hbm, out_hbm,
        acc_sh, src_v, dst_v, rows_v, sem):
    c = lax.axis_index("c")
    s = lax.axis_index("s")
    wid = s * NC + c

    pltpu.sync_copy(zer_hbm, acc_sh.at[pl.ds(s * RPT, RPT)])
    plsc.subcore_barrier()

    base = wid * EPW

    def step(i, carry):
        off = base + i * CH
        pltpu.sync_copy(src_hbm.at[pl.ds(off, CH)], src_v)
        pltpu.sync_copy(dst_hbm.at[pl.ds(off, CH)], dst_v)
        pltpu.async_copy(h_hbm.at[src_v], rows_v, sem).wait()
        pltpu.sync_copy(rows_v, acc_sh.at[dst_v], add=True)
        return carry

    lax.fori_loop(0, NCHUNK, step, 0)
    plsc.subcore_barrier()

    pltpu.sync_copy(acc_sh.at[pl.ds(s * RPT, RPT)],
                    out_hbm.at[c].at[pl.ds(s * RPT, RPT)])


def _mlp_body(h_ref, p0_ref, p1_ref, w1_ref, b1_ref, w2_ref, b2_ref, o_ref):
    m = h_ref[...] + p0_ref[0] + p1_ref[0]
    t = jnp.maximum(
        jnp.dot(m, w1_ref[...], preferred_element_type=jnp.float32)
        + b1_ref[...], 0.0)
    o_ref[...] = jnp.maximum(
        jnp.dot(t, w2_ref[...], preferred_element_type=jnp.float32)
        + b2_ref[...], 0.0)


_mlp = pl.pallas_call(
    _mlp_body,
    grid=(NBLK,),
    in_specs=[
        pl.BlockSpec((R, H), lambda i: (i, 0)),
        pl.BlockSpec((1, R, H), lambda i: (0, i, 0)),
        pl.BlockSpec((1, R, H), lambda i: (1, i, 0)),
        pl.BlockSpec((H, H), lambda i: (0, 0)),
        pl.BlockSpec((1, H), lambda i: (0, 0)),
        pl.BlockSpec((H, H), lambda i: (0, 0)),
        pl.BlockSpec((1, H), lambda i: (0, 0)),
    ],
    out_specs=pl.BlockSpec((R, H), lambda i: (i, 0)),
    out_shape=jax.ShapeDtypeStruct((N, H), jnp.float32),
)


def _final_body(h1_ref, h2_ref, p0_ref, p1_ref, batch_ref,
                w1_ref, b1_ref, w2_ref, b2_ref,
                wa_ref, wb_ref, wc_ref, bjk_ref,
                wc1_ref, bc1_ref, gamma_ref, beta_ref, wc2_ref, bc2_ref,
                o_ref, g_acc):
    i = pl.program_id(0)

    m = h2_ref[...] + p0_ref[0] + p1_ref[0]
    t = jnp.maximum(
        jnp.dot(m, w1_ref[...], preferred_element_type=jnp.float32)
        + b1_ref[...], 0.0)
    h3 = jnp.maximum(
        jnp.dot(t, w2_ref[...], preferred_element_type=jnp.float32)
        + b2_ref[...], 0.0)

    hjk = (jnp.dot(h1_ref[...], wa_ref[...], preferred_element_type=jnp.float32)
           + jnp.dot(h2_ref[...], wb_ref[...], preferred_element_type=jnp.float32)
           + jnp.dot(h3, wc_ref[...], preferred_element_type=jnp.float32)
           + bjk_ref[...])

    b = batch_ref[0, 0, :]
    onehot = (b[:, None] ==
              lax.broadcasted_iota(jnp.int32, (R, G), 1)).astype(jnp.float32)
    part = lax.dot_general(onehot, hjk, (((0,), (0,)), ((), ())),
                           preferred_element_type=jnp.float32,
                           precision=lax.Precision.HIGHEST)

    @pl.when(i == 0)
    def _():
        g_acc[...] = jnp.zeros_like(g_acc)

    g_acc[...] += part

    @pl.when(i == NBLK - 1)
    def _():
        z = (jnp.dot(g_acc[...], wc1_ref[...],
                     preferred_element_type=jnp.float32) + bc1_ref[...])
        mu = jnp.mean(z, axis=0, keepdims=True)
        var = jnp.mean((z - mu) ** 2, axis=0, keepdims=True)
        z = (z - mu) * lax.rsqrt(var + 1e-5) * gamma_ref[...] + beta_ref[...]
        z = jnp.maximum(z, 0.0)
        o_ref[...] = (jnp.dot(z, wc2_ref[...],
                              preferred_element_type=jnp.float32)
                      + bc2_ref[...])


_final = pl.pallas_call(
    _final_body,
    grid=(NBLK,),
    in_specs=[
        pl.BlockSpec((R, H), lambda i: (i, 0)),
        pl.BlockSpec((R, H), lambda i: (i, 0)),
        pl.BlockSpec((1, R, H), lambda i: (0, i, 0)),
        pl.BlockSpec((1, R, H), lambda i: (1, i, 0)),
        pl.BlockSpec((1, 1, R), lambda i: (i, 0, 0)),
        pl.BlockSpec((H, H), lambda i: (0, 0)),
        pl.BlockSpec((1, H), lambda i: (0, 0)),
        pl.BlockSpec((H, H), lambda i: (0, 0)),
        pl.BlockSpec((1, H), lambda i: (0, 0)),
        pl.BlockSpec((H, H), lambda i: (0, 0)),
        pl.BlockSpec((H, H), lambda i: (0, 0)),
        pl.BlockSpec((H, H), lambda i: (0, 0)),
        pl.BlockSpec((1, H), lambda i: (0, 0)),
        pl.BlockSpec((H, 2 * H), lambda i: (0, 0)),
        pl.BlockSpec((1, 2 * H), lambda i: (0, 0)),
        pl.BlockSpec((1, 2 * H), lambda i: (0, 0)),
        pl.BlockSpec((1, 2 * H), lambda i: (0, 0)),
        pl.BlockSpec((2 * H, H), lambda i: (0, 0)),
        pl.BlockSpec((1, H), lambda i: (0, 0)),
    ],
    out_specs=pl.BlockSpec((G, H), lambda i: (0, 0)),
    out_shape=jax.ShapeDtypeStruct((G, H), jnp.float32),
    scratch_shapes=[pltpu.VMEM((G, H), jnp.float32)],
)


def kernel(x, edge_index, batch,
           W1_0, b1_0, W2_0, b2_0,
           W1_1, b1_1, W2_1, b2_1,
           W1_2, b1_2, W2_2, b2_2,
           W_jk, b_jk, Wc1, bc1, gamma, beta, Wc2, bc2):
    order = jnp.argsort(edge_index[1], stable=True)
    src = edge_index[0][order]
    dst = edge_index[1][order]
    zer = jnp.zeros((RPT, H), jnp.float32)
    batch3 = batch.reshape(NBLK, 1, R)

    Wa = W_jk[:H]
    Wb = W_jk[H:2 * H]
    Wc = W_jk[2 * H:]
    ncls = Wc2.shape[1]
    Wc2p = jnp.zeros((2 * H, H), jnp.float32).at[:, :ncls].set(Wc2)
    bc2p = jnp.zeros((1, H), jnp.float32).at[0, :ncls].set(bc2)

    h = x
    parts = _mp(src, dst, zer, h)
    h1 = _mlp(h, parts, parts, W1_0, b1_0.reshape(1, H), W2_0,
              b2_0.reshape(1, H))
    parts = _mp(src, dst, zer, h1)
    h2 = _mlp(h1, parts, parts, W1_1, b1_1.reshape(1, H), W2_1,
              b2_1.reshape(1, H))
    parts = _mp(src, dst, zer, h2)
    out = _final(h1, h2, parts, parts, batch3,
                 W1_2, b1_2.reshape(1, H), W2_2, b2_2.reshape(1, H),
                 Wa, Wb, Wc, b_jk.reshape(1, H),
                 Wc1, bc1.reshape(1, 2 * H), gamma.reshape(1, 2 * H),
                 beta.reshape(1, 2 * H), Wc2p, bc2p)
    return out[:, :ncls]
